# Initial kernel scaffold; baseline (speedup 1.0000x reference)
import jax
import jax.numpy as jnp
from jax.experimental import pallas as pl


def kernel(h, triplets, W):
    raise NotImplementedError("write your pallas kernel here")


# speedup vs baseline: 4.7305x; 4.7305x over previous
import functools

import jax
import jax.numpy as jnp
from jax import lax
from jax.experimental import pallas as pl
from jax.experimental.pallas import tpu as pltpu
from jax.experimental.pallas import tpu_sc as plsc

_LANES = 16
_NC = 2
_NS = 16
_NW = _NC * _NS
_C = 128


def _make_sc_call(n_triplets: int, n_rows_h: int, n_rows_w: int, feat: int):
    assert feat % _LANES == 0
    per_w = n_triplets // _NW
    assert per_w * _NW == n_triplets
    assert per_w % _LANES == 0
    n_full = per_w // _C
    tail = per_w - n_full * _C
    d_chunks = feat // _LANES

    mesh = plsc.VectorSubcoreMesh(core_axis_name="c", subcore_axis_name="s")

    def body(h_hbm, s_hbm, r_hbm, o_hbm, w_hbm, out_hbm,
             s_idx, r_idx, o_idx, hs, wr, ho, tile, out_v, sem):
        wid = lax.axis_index("s") * _NC + lax.axis_index("c")
        base = wid * per_w

        pltpu.sync_copy(s_hbm.at[pl.ds(base, per_w)], s_idx)
        pltpu.sync_copy(r_hbm.at[pl.ds(base, per_w)], r_idx)
        pltpu.sync_copy(o_hbm.at[pl.ds(base, per_w)], o_idx)

        lane = lax.iota(jnp.int32, _LANES)

        def compute_groups(off, n_groups):
            def group(g, carry):
                for j in range(_LANES):
                    row = g * _LANES + j
                    acc = (hs[row, pl.ds(0, _LANES)]
                           * wr[row, pl.ds(0, _LANES)]
                           * ho[row, pl.ds(0, _LANES)])
                    for d in range(1, d_chunks):
                        acc = acc + (hs[row, pl.ds(d * _LANES, _LANES)]
                                     * wr[row, pl.ds(d * _LANES, _LANES)]
                                     * ho[row, pl.ds(d * _LANES, _LANES)])
                    tile[j, :] = acc
                scores = jnp.zeros((_LANES,), jnp.float32)
                for d in range(_LANES):
                    col = jnp.full((_LANES,), d, jnp.int32)
                    scores = scores + plsc.load_gather(tile, [lane, col])
                out_v[pl.ds(off + g * _LANES, _LANES)] = scores
                return carry
            lax.fori_loop(0, n_groups, group, 0)

        def gather_chunk(off, n):
            c1 = pltpu.async_copy(h_hbm.at[s_idx.at[pl.ds(off, n)]],
                                  hs.at[pl.ds(0, n)], sem)
            c2 = pltpu.async_copy(w_hbm.at[r_idx.at[pl.ds(off, n)]],
                                  wr.at[pl.ds(0, n)], sem)
            c3 = pltpu.async_copy(h_hbm.at[o_idx.at[pl.ds(off, n)]],
                                  ho.at[pl.ds(0, n)], sem)
            c1.wait()
            c2.wait()
            c3.wait()

        def chunk(c, carry):
            off = c * _C
            gather_chunk(off, _C)
            compute_groups(off, _C // _LANES)
            return carry
        lax.fori_loop(0, n_full, chunk, 0)

        if tail:
            off = n_full * _C
            gather_chunk(off, tail)
            compute_groups(off, tail // _LANES)

        pltpu.sync_copy(out_v, out_hbm.at[pl.ds(base, per_w)])

    return functools.partial(
        pl.kernel,
        out_type=jax.ShapeDtypeStruct((n_triplets,), jnp.float32),
        mesh=mesh,
        compiler_params=pltpu.CompilerParams(needs_layout_passes=False),
        scratch_types=[
            pltpu.VMEM((per_w,), jnp.int32),
            pltpu.VMEM((per_w,), jnp.int32),
            pltpu.VMEM((per_w,), jnp.int32),
            pltpu.VMEM((_C, feat), jnp.float32),
            pltpu.VMEM((_C, feat), jnp.float32),
            pltpu.VMEM((_C, feat), jnp.float32),
            pltpu.VMEM((_LANES, _LANES), jnp.float32),
            pltpu.VMEM((per_w,), jnp.float32),
            pltpu.SemaphoreType.DMA,
        ],
    )(body)


def kernel(h, triplets, W):
    n_triplets = triplets.shape[0]
    call = _make_sc_call(n_triplets, h.shape[0], W.shape[0], h.shape[1])
    s = triplets[:, 0]
    r = triplets[:, 1]
    o = triplets[:, 2]
    return call(h, s, r, o, W)


# speedup vs baseline: 7.8421x; 1.6578x over previous
import functools

import jax
import jax.numpy as jnp
from jax import lax
from jax.experimental import pallas as pl
from jax.experimental.pallas import tpu as pltpu
from jax.experimental.pallas import tpu_sc as plsc

_LANES = 16
_NC = 2
_NS = 16
_NW = _NC * _NS
_C = 80


def _make_sc_call(n_triplets: int, feat: int):
    assert feat % _LANES == 0
    per_w = n_triplets // _NW
    assert per_w * _NW == n_triplets
    assert per_w % _C == 0
    n_chunks = per_w // _C
    d_chunks = feat // _LANES

    mesh = plsc.VectorSubcoreMesh(core_axis_name="c", subcore_axis_name="s")

    def body(h_hbm, s_hbm, r_hbm, o_hbm, w_hbm, out_hbm,
             s_idx, r_idx, o_idx, hs, wr, ho, tile, out_v, semg):
        wid = lax.axis_index("s") * _NC + lax.axis_index("c")
        base = wid * per_w

        pltpu.sync_copy(s_hbm.at[pl.ds(base, per_w)], s_idx)
        pltpu.sync_copy(r_hbm.at[pl.ds(base, per_w)], r_idx)
        pltpu.sync_copy(o_hbm.at[pl.ds(base, per_w)], o_idx)

        lane = lax.iota(jnp.int32, _LANES)

        def issue(c, b):
            off = c * _C
            pltpu.async_copy(h_hbm.at[s_idx.at[pl.ds(off, _C)]],
                             hs.at[b], semg.at[b])
            pltpu.async_copy(w_hbm.at[r_idx.at[pl.ds(off, _C)]],
                             wr.at[b], semg.at[b])
            pltpu.async_copy(h_hbm.at[o_idx.at[pl.ds(off, _C)]],
                             ho.at[b], semg.at[b])

        def wait3(b):
            pltpu.make_async_copy(h_hbm.at[s_idx.at[pl.ds(0, _C)]],
                                  hs.at[b], semg.at[b]).wait()
            pltpu.make_async_copy(w_hbm.at[r_idx.at[pl.ds(0, _C)]],
                                  wr.at[b], semg.at[b]).wait()
            pltpu.make_async_copy(h_hbm.at[o_idx.at[pl.ds(0, _C)]],
                                  ho.at[b], semg.at[b]).wait()

        def compute(c, b):
            off = c * _C

            def group(g, carry):
                for j in range(_LANES):
                    row = g * _LANES + j
                    acc = (hs[b, row, pl.ds(0, _LANES)]
                           * wr[b, row, pl.ds(0, _LANES)]
                           * ho[b, row, pl.ds(0, _LANES)])
                    for d in range(1, d_chunks):
                        acc = acc + (hs[b, row, pl.ds(d * _LANES, _LANES)]
                                     * wr[b, row, pl.ds(d * _LANES, _LANES)]
                                     * ho[b, row, pl.ds(d * _LANES, _LANES)])
                    tile[j, :] = acc
                scores = jnp.zeros((_LANES,), jnp.float32)
                for d in range(_LANES):
                    col = jnp.full((_LANES,), d, jnp.int32)
                    scores = scores + plsc.load_gather(tile, [lane, col])
                out_v[pl.ds(off + g * _LANES, _LANES)] = scores
                return carry

            lax.fori_loop(0, _C // _LANES, group, 0)

        issue(0, 0)

        def step(c, carry):
            issue(c + 1, (c + 1) & 1)
            wait3(c & 1)
            compute(c, c & 1)
            return carry

        lax.fori_loop(0, n_chunks - 1, step, 0)
        last = n_chunks - 1
        wait3(last & 1)
        compute(last, last & 1)

        pltpu.sync_copy(out_v, out_hbm.at[pl.ds(base, per_w)])

    return functools.partial(
        pl.kernel,
        out_type=jax.ShapeDtypeStruct((n_triplets,), jnp.float32),
        mesh=mesh,
        compiler_params=pltpu.CompilerParams(needs_layout_passes=False),
        scratch_types=[
            pltpu.VMEM((per_w,), jnp.int32),
            pltpu.VMEM((per_w,), jnp.int32),
            pltpu.VMEM((per_w,), jnp.int32),
            pltpu.VMEM((2, _C, feat), jnp.float32),
            pltpu.VMEM((2, _C, feat), jnp.float32),
            pltpu.VMEM((2, _C, feat), jnp.float32),
            pltpu.VMEM((_LANES, _LANES), jnp.float32),
            pltpu.VMEM((per_w,), jnp.float32),
            pltpu.SemaphoreType.DMA((2,)),
        ],
    )(body)


def kernel(h, triplets, W):
    n_triplets = triplets.shape[0]
    call = _make_sc_call(n_triplets, h.shape[1])
    s = triplets[:, 0]
    r = triplets[:, 1]
    o = triplets[:, 2]
    return call(h, s, r, o, W)


# speedup vs baseline: 8.1119x; 1.0344x over previous
import functools

import jax
import jax.numpy as jnp
from jax import lax
from jax.experimental import pallas as pl
from jax.experimental.pallas import tpu as pltpu
from jax.experimental.pallas import tpu_sc as plsc

_LANES = 16
_NC = 2
_NS = 16
_NW = _NC * _NS
_C = 80


def _make_sc_call(n_triplets: int, feat: int):
    assert feat % (2 * _LANES) == 0
    fw = feat // 2
    per_w = n_triplets // _NW
    assert per_w * _NW == n_triplets
    assert per_w % _C == 0
    n_chunks = per_w // _C
    d_chunks = fw // _LANES

    mesh = plsc.VectorSubcoreMesh(core_axis_name="c", subcore_axis_name="s")

    def body(h_hbm, s_hbm, r_hbm, o_hbm, w_hbm, out_hbm,
             s_idx, r_idx, o_idx, hs, wr, ho, tile, out_v, semg):
        wid = lax.axis_index("s") * _NC + lax.axis_index("c")
        base = wid * per_w

        pltpu.sync_copy(s_hbm.at[pl.ds(base, per_w)], s_idx)
        pltpu.sync_copy(r_hbm.at[pl.ds(base, per_w)], r_idx)
        pltpu.sync_copy(o_hbm.at[pl.ds(base, per_w)], o_idx)

        lane = lax.iota(jnp.int32, _LANES)

        def issue(c, b):
            off = c * _C
            pltpu.async_copy(h_hbm.at[s_idx.at[pl.ds(off, _C)]],
                             hs.at[b], semg.at[b])
            pltpu.async_copy(w_hbm.at[r_idx.at[pl.ds(off, _C)]],
                             wr.at[b], semg.at[b])
            pltpu.async_copy(h_hbm.at[o_idx.at[pl.ds(off, _C)]],
                             ho.at[b], semg.at[b])

        def wait3(b):
            pltpu.make_async_copy(h_hbm.at[s_idx.at[pl.ds(0, _C)]],
                                  hs.at[b], semg.at[b]).wait()
            pltpu.make_async_copy(w_hbm.at[r_idx.at[pl.ds(0, _C)]],
                                  wr.at[b], semg.at[b]).wait()
            pltpu.make_async_copy(h_hbm.at[o_idx.at[pl.ds(0, _C)]],
                                  ho.at[b], semg.at[b]).wait()

        def compute(c, b):
            off = c * _C

            def group(g, carry):
                for j in range(_LANES):
                    row = g * _LANES + j
                    acc = jnp.zeros((_LANES,), jnp.float32)
                    for d in range(d_chunks):
                        sl = pl.ds(d * _LANES, _LANES)
                        a0, a1 = plsc.unpack(
                            plsc.bitcast(hs[b, row, sl], jnp.bfloat16),
                            format=plsc.PackFormat.INTERLEAVED,
                            preferred_element_type=jnp.float32)
                        b0, b1 = plsc.unpack(
                            plsc.bitcast(wr[b, row, sl], jnp.bfloat16),
                            format=plsc.PackFormat.INTERLEAVED,
                            preferred_element_type=jnp.float32)
                        c0, c1 = plsc.unpack(
                            plsc.bitcast(ho[b, row, sl], jnp.bfloat16),
                            format=plsc.PackFormat.INTERLEAVED,
                            preferred_element_type=jnp.float32)
                        acc = acc + a0 * b0 * c0
                        acc = acc + a1 * b1 * c1
                    tile[j, :] = acc
                scores = jnp.zeros((_LANES,), jnp.float32)
                for d in range(_LANES):
                    col = jnp.full((_LANES,), d, jnp.int32)
                    scores = scores + plsc.load_gather(tile, [lane, col])
                out_v[pl.ds(off + g * _LANES, _LANES)] = scores
                return carry

            lax.fori_loop(0, _C // _LANES, group, 0)

        issue(0, 0)

        def step(c, carry):
            issue(c + 1, (c + 1) & 1)
            wait3(c & 1)
            compute(c, c & 1)
            return carry

        lax.fori_loop(0, n_chunks - 1, step, 0)
        last = n_chunks - 1
        wait3(last & 1)
        compute(last, last & 1)

        pltpu.sync_copy(out_v, out_hbm.at[pl.ds(base, per_w)])

    return functools.partial(
        pl.kernel,
        out_type=jax.ShapeDtypeStruct((n_triplets,), jnp.float32),
        mesh=mesh,
        compiler_params=pltpu.CompilerParams(
            needs_layout_passes=False, use_tc_tiling_on_sc=False),
        scratch_types=[
            pltpu.VMEM((per_w,), jnp.int32),
            pltpu.VMEM((per_w,), jnp.int32),
            pltpu.VMEM((per_w,), jnp.int32),
            pltpu.VMEM((2, _C, fw), jnp.int32),
            pltpu.VMEM((2, _C, fw), jnp.int32),
            pltpu.VMEM((2, _C, fw), jnp.int32),
            pltpu.VMEM((_LANES, _LANES), jnp.float32),
            pltpu.VMEM((per_w,), jnp.float32),
            pltpu.SemaphoreType.DMA((2,)),
        ],
    )(body)


def kernel(h, triplets, W):
    n_triplets = triplets.shape[0]
    call = _make_sc_call(n_triplets, h.shape[1])
    s = triplets[:, 0]
    r = triplets[:, 1]
    o = triplets[:, 2]

    def pack32(x):
        x16 = x.astype(jnp.bfloat16)
        return lax.bitcast_convert_type(
            x16.reshape(x.shape[0], x.shape[1] // 2, 2), jnp.int32)

    return call(pack32(h), s, r, o, pack32(W))
